# baseline (speedup 1.0000x reference)
import functools

import jax
import jax.numpy as jnp
from jax import lax
from jax.experimental import pallas as pl
from jax.experimental.pallas import tpu as pltpu
from jax.experimental.pallas import tpu_sc as plsc

H = 128
L = 16
NCH = H // L
NSUB = 16
NCORE = 2
CHUNKS_PER_CORE = NCH // NCORE

_N = {'operator': 100000, 'table': 50000, 'column': 100000, 'predicate': 80000}
_ETYPES = [('table', 'operator', 'scannedby'),
           ('predicate', 'operator', 'filters'),
           ('column', 'operator', 'outputby'),
           ('column', 'predicate', 'connects'),
           ('operator', 'operator', 'calledby'),
           ('table', 'table', 'selfloop_table'),
           ('column', 'column', 'selfloop_column')]
_ECNT = {'scannedby': 100000, 'filters': 100000, 'outputby': 100000,
         'connects': 100000, 'calledby': 100000, 'selfloop_table': 50000,
         'selfloop_column': 100000}
NUM_GRAPHS = 64
_NTYPES = ['operator', 'table', 'column', 'predicate']


def _ceil_to(x, m):
    return (x + m - 1) // m * m


def _bf(a):
    return a.astype(jnp.bfloat16)


def _bdot(a_bf, b_bf):
    return jnp.dot(a_bf, b_bf, preferred_element_type=jnp.float32)


_NPAD = {t: _ceil_to(_N[t] + 1, 2048) for t in _NTYPES}
_EPAD = {n: _ceil_to(_ECNT[n], 2048) for n in _ECNT}
_ACC_ROWS = max(_NPAD.values())
_EROWS_MAX = _ceil_to(max(_EPAD.values()) // (NSUB * 128), 8)


def _inner_k(nrows):
    for k in (8, 7, 5, 6, 4, 3, 2):
        if nrows % k == 0:
            return k
    return 1


def _combine(xs, dots, bias, relu_out=True, bn=2048):
    np_rows = xs[0].shape[0]
    grid = (np_rows // bn,)
    k = len(xs)

    def body(*refs):
        x_refs = refs[:k]
        w_refs = refs[k:k + len(dots)]
        b_ref = refs[k + len(dots)]
        o_ref = refs[k + len(dots) + 1]
        xb = [_bf(x_ref[...]) for x_ref in x_refs]
        y = jnp.broadcast_to(b_ref[...], (bn, H))
        for (xi, _), w_ref in zip(dots, w_refs):
            y = y + _bdot(xb[xi], w_ref[...])
        if relu_out:
            y = jnp.maximum(y, 0.0)
        o_ref[...] = y

    return pl.pallas_call(
        body, grid=grid,
        in_specs=([pl.BlockSpec((bn, x.shape[1]), lambda i: (i, 0))
                   for x in xs]
                  + [pl.BlockSpec(w.shape, lambda i: (0, 0))
                     for _, w in dots]
                  + [pl.BlockSpec((1, H), lambda i: (0, 0))]),
        out_specs=pl.BlockSpec((bn, H), lambda i: (i, 0)),
        out_shape=jax.ShapeDtypeStruct((np_rows, H), jnp.float32),
    )(*xs, *[w for _, w in dots], bias)


def _pool_combine(xs, dots, bias, ids3, wt, b, bn=2048):
    np_rows = xs[0].shape[0]
    ngrid = np_rows // bn
    k = len(xs)

    def body(*refs):
        x_refs = refs[:k]
        w_refs = refs[k:k + len(dots)]
        b_ref = refs[k + len(dots)]
        ids_ref = refs[k + len(dots) + 1]
        wo_ref = refs[k + len(dots) + 2]
        bo_ref = refs[k + len(dots) + 3]
        o_ref = refs[k + len(dots) + 4]
        acc = refs[k + len(dots) + 5]
        cnt = refs[k + len(dots) + 6]
        i = pl.program_id(0)

        @pl.when(i == 0)
        def _():
            acc[...] = jnp.zeros_like(acc)
            cnt[...] = jnp.zeros_like(cnt)

        xb = [_bf(x_ref[...]) for x_ref in x_refs]
        y = jnp.broadcast_to(b_ref[...], (bn, H))
        for (xi, _), w_ref in zip(dots, w_refs):
            y = y + _bdot(xb[xi], w_ref[...])
        y = jnp.maximum(y, 0.0)
        ids = ids_ref[0]
        gids = lax.broadcasted_iota(jnp.int32, (NUM_GRAPHS, bn), 0)
        oh = (gids == ids).astype(jnp.float32)
        acc[...] += jnp.dot(oh, y, precision=lax.Precision.HIGHEST)
        cnt[...] += jnp.sum(oh, axis=1, keepdims=True)

        @pl.when(i == ngrid - 1)
        def _():
            pooled = acc[...] / jnp.maximum(cnt[...], 1.0)
            o_ref[...] = _bdot(_bf(pooled), _bf(wo_ref[...])) + bo_ref[...]

    return pl.pallas_call(
        body, grid=(ngrid,),
        in_specs=([pl.BlockSpec((bn, x.shape[1]), lambda i: (i, 0))
                   for x in xs]
                  + [pl.BlockSpec(w.shape, lambda i: (0, 0))
                     for _, w in dots]
                  + [pl.BlockSpec((1, H), lambda i: (0, 0)),
                     pl.BlockSpec((1, 1, bn), lambda i: (i, 0, 0)),
                     pl.BlockSpec(wt.shape, lambda i: (0, 0)),
                     pl.BlockSpec((1, 1), lambda i: (0, 0))]),
        out_specs=pl.BlockSpec((NUM_GRAPHS, 1), lambda i: (0, 0)),
        out_shape=jax.ShapeDtypeStruct((NUM_GRAPHS, 1), jnp.float32),
        scratch_shapes=[
            pltpu.VMEM((NUM_GRAPHS, H), jnp.float32),
            pltpu.VMEM((NUM_GRAPHS, 1), jnp.float32),
        ],
    )(*xs, *[w for _, w in dots], bias, ids3, wt, b)


def _sc_layer(dst_specs, rel_erows, zero, zs, srcs, dsts):
    ntypes = len(dst_specs)
    nrels = len(zs)
    mesh = plsc.VectorSubcoreMesh(core_axis_name="c", subcore_axis_name="s",
                                  num_cores=NCORE, num_subcores=NSUB)

    @functools.partial(
        pl.kernel,
        out_type=[jax.ShapeDtypeStruct((sp[0], H), jnp.float32)
                  for sp in dst_specs],
        mesh=mesh,
        scratch_types=[
            pltpu.VMEM_SHARED((_ACC_ROWS, L), jnp.float32),
            pltpu.VMEM((_EROWS_MAX, 128), jnp.int32),
            pltpu.VMEM((_EROWS_MAX, 128), jnp.int32),
            pltpu.VMEM((8 * 128, L), jnp.float32),
            pltpu.SemaphoreType.DMA,
            pltpu.SemaphoreType.DMA,
        ],
        compiler_params=pltpu.CompilerParams(use_tc_tiling_on_sc=False),
    )
    def kfn(*refs):
        zero_ref = refs[0]
        z_refs = refs[1:1 + nrels]
        s_refs = refs[1 + nrels:1 + 2 * nrels]
        d_refs = refs[1 + 2 * nrels:1 + 3 * nrels]
        out_refs = refs[1 + 3 * nrels:1 + 3 * nrels + ntypes]
        acc, sidx, didx, rows, gsem, ssem = refs[1 + 3 * nrels + ntypes:]

        cid = lax.axis_index("c")
        sid = lax.axis_index("s")

        for ti, (nt_pad, rel_ids) in enumerate(dst_specs):
            rpw = nt_pad // NSUB
            for cc in range(CHUNKS_PER_CORE):
                ch = cid * CHUNKS_PER_CORE + cc
                pltpu.sync_copy(zero_ref.at[pl.ds(0, rpw)],
                                acc.at[pl.ds(sid * rpw, rpw)])
                plsc.subcore_barrier()
                for r in rel_ids:
                    erows = rel_erows[r]
                    stride = s_refs[r].shape[1] // NSUB
                    ki = _inner_k(erows)
                    nouter = erows // ki
                    pltpu.sync_copy(
                        s_refs[r].at[ch].at[pl.ds(sid * stride, stride)],
                        sidx.at[pl.ds(0, stride)])
                    pltpu.sync_copy(d_refs[r].at[pl.ds(sid * stride, stride)],
                                    didx.at[pl.ds(0, stride)])

                    def outer(o, _, r=r, ki=ki):
                        gds, sds = [], []
                        for j in range(ki):
                            gds.append(pltpu.async_copy(
                                z_refs[r].at[sidx.at[o * ki + j]],
                                rows.at[pl.ds(j * 128, 128)], gsem))
                        for dsc in gds:
                            dsc.wait()
                        for j in range(ki):
                            sds.append(pltpu.async_copy(
                                rows.at[pl.ds(j * 128, 128)],
                                acc.at[didx.at[o * ki + j]], ssem, add=True))
                        for dsc in sds:
                            dsc.wait()
                        return 0

                    lax.fori_loop(0, nouter, outer, 0)
                plsc.subcore_barrier()
                pltpu.sync_copy(
                    acc.at[pl.ds(sid * rpw, rpw)],
                    out_refs[ti].at[pl.ds(sid * rpw, rpw), pl.ds(ch * L, L)])
                plsc.subcore_barrier()

    return kfn(zero, *zs, *srcs, *dsts)


def kernel(x_operator, x_table, x_column, x_predicate, params,
           edge_index_scannedby, edge_index_filters, edge_index_outputby,
           edge_index_connects, edge_index_calledby,
           edge_index_selfloop_table, edge_index_selfloop_column,
           batch_operator):
    xs = {'operator': x_operator, 'table': x_table,
          'column': x_column, 'predicate': x_predicate}
    edges = {'scannedby': edge_index_scannedby, 'filters': edge_index_filters,
             'outputby': edge_index_outputby, 'connects': edge_index_connects,
             'calledby': edge_index_calledby,
             'selfloop_table': edge_index_selfloop_table,
             'selfloop_column': edge_index_selfloop_column}
    p = params

    xp = {t: jnp.pad(xs[t], ((0, _NPAD[t] - _N[t]), (0, 0))) for t in _NTYPES}
    srcp, dstp, erows_d = {}, {}, {}
    for (st, dt, name) in _ETYPES:
        e = _ECNT[name]
        ep = _EPAD[name]
        erows = ep // (NSUB * 128)
        stride = _ceil_to(erows, 8)
        erows_d[name] = erows

        def _lay(v, fill):
            v = jnp.pad(v, (0, ep - e), constant_values=fill)
            v = v.reshape(NSUB, erows, 128)
            v = jnp.pad(v, ((0, 0), (0, stride - erows), (0, 0)),
                        constant_values=fill)
            return v.reshape(NSUB * stride, 128)

        s0 = _lay(edges[name][0], 0)
        srcp[name] = (s0[None] * NCH
                      + jnp.arange(NCH, dtype=jnp.int32)[:, None, None])
        dstp[name] = _lay(edges[name][1], _N[dt])

    max_rpw = max(_NPAD[t] for t in _NTYPES) // NSUB
    zero = jnp.zeros((max_rpw, L), jnp.float32)
    rel_order = [name for (_, _, name) in _ETYPES]
    erows_l = [erows_d[name] for name in rel_order]

    h0 = {}
    for t in _NTYPES:
        h0[t] = _combine([xp[t]], [(0, _bf(p['lin_%s_W' % t].T))],
                         p['lin_%s_b' % t].reshape(1, H), relu_out=False)

    dst_specs1 = [(_NPAD[dt], [ri]) for ri, (st, dt, name)
                  in enumerate(_ETYPES)]
    aggs1 = _sc_layer(dst_specs1, erows_l, zero,
                      [h0[st].reshape(_NPAD[st] * NCH, L)
                       for (st, dt, name) in _ETYPES],
                      [srcp[name] for name in rel_order],
                      [dstp[name] for name in rel_order])
    agg1 = dict(zip(rel_order, aggs1))

    out1 = {}
    for t in _NTYPES:
        rels_t = [name for (st, dt, name) in _ETYPES if dt == t]
        ins = [agg1[name] for name in rels_t] + [h0[t]]
        dots = [(i, _bf(p['c1_%s_Wrel' % name].T))
                for i, name in enumerate(rels_t)]
        dots += [(len(rels_t), _bf(p['c1_%s_Wroot' % name].T))
                 for name in rels_t]
        bias = sum(p['c1_%s_brel' % name] for name in rels_t).reshape(1, H)
        out1[t] = _combine(ins, dots, bias, relu_out=True)

    l2_rels = [(st, dt, name) for (st, dt, name) in _ETYPES if dt == 'operator']
    dst_specs2 = [(_NPAD['operator'], [ri]) for ri in range(len(l2_rels))]
    erows2 = [erows_d[name] for (_, _, name) in l2_rels]
    aggs2 = _sc_layer(dst_specs2, erows2, zero,
                      [out1[st].reshape(_NPAD[st] * NCH, L)
                       for (st, dt, name) in l2_rels],
                      [srcp[name] for (_, _, name) in l2_rels],
                      [dstp[name] for (_, _, name) in l2_rels])

    ins2 = list(aggs2) + [out1['operator']]
    dots2 = [(i, _bf(p['c2_%s_Wrel' % name].T))
             for i, (_, _, name) in enumerate(l2_rels)]
    dots2 += [(len(l2_rels), _bf(p['c2_%s_Wroot' % name].T))
              for (_, _, name) in l2_rels]
    bias2 = sum(p['c2_%s_brel' % name] for (_, _, name) in l2_rels
                ).reshape(1, H)
    ids = jnp.pad(batch_operator, (0, _NPAD['operator'] - _N['operator']),
                  constant_values=NUM_GRAPHS + 1)
    ids3 = ids.reshape(_NPAD['operator'] // 2048, 1, 2048)
    res = _pool_combine(ins2, dots2, bias2, ids3, p['lin_out_W'].T,
                        p['lin_out_b'].reshape(1, 1))
    return res.reshape(NUM_GRAPHS)


# speedup vs baseline: 1.0281x; 1.0281x over previous
import functools

import jax
import jax.numpy as jnp
from jax import lax
from jax.experimental import pallas as pl
from jax.experimental.pallas import tpu as pltpu
from jax.experimental.pallas import tpu_sc as plsc

H = 128
L = 16
NCH = H // L
NSUB = 16
NCORE = 2
CHUNKS_PER_CORE = NCH // NCORE

_N = {'operator': 100000, 'table': 50000, 'column': 100000, 'predicate': 80000}
_ETYPES = [('table', 'operator', 'scannedby'),
           ('predicate', 'operator', 'filters'),
           ('column', 'operator', 'outputby'),
           ('column', 'predicate', 'connects'),
           ('operator', 'operator', 'calledby'),
           ('table', 'table', 'selfloop_table'),
           ('column', 'column', 'selfloop_column')]
_ECNT = {'scannedby': 100000, 'filters': 100000, 'outputby': 100000,
         'connects': 100000, 'calledby': 100000, 'selfloop_table': 50000,
         'selfloop_column': 100000}
NUM_GRAPHS = 64
_NTYPES = ['operator', 'table', 'column', 'predicate']


def _ceil_to(x, m):
    return (x + m - 1) // m * m


def _bf(a):
    return a.astype(jnp.bfloat16)


def _bdot(a_bf, b_bf):
    return jnp.dot(a_bf, b_bf, preferred_element_type=jnp.float32)


_NPAD = {t: _ceil_to(_N[t] + 1, 2048) for t in _NTYPES}
_EPAD = {n: _ceil_to(_ECNT[n], 2048) for n in _ECNT}
_ACC_ROWS = max(_NPAD.values())
_EROWS_MAX = _ceil_to(max(_EPAD.values()) // (NSUB * 128), 8)


def _inner_k(nrows):
    for k in (8, 7, 5, 6, 4, 3, 2):
        if nrows % k == 0:
            return k
    return 1


def _combine(xs, dots, bias, relu_out=True, bn=2048):
    np_rows = xs[0].shape[0]
    grid = (np_rows // bn,)
    k = len(xs)

    def body(*refs):
        x_refs = refs[:k]
        w_refs = refs[k:k + len(dots)]
        b_ref = refs[k + len(dots)]
        o_ref = refs[k + len(dots) + 1]
        xb = [_bf(x_ref[...]) for x_ref in x_refs]
        y = jnp.broadcast_to(b_ref[...], (bn, H))
        for (xi, _), w_ref in zip(dots, w_refs):
            y = y + _bdot(xb[xi], w_ref[...])
        if relu_out:
            y = jnp.maximum(y, 0.0)
        o_ref[...] = y

    return pl.pallas_call(
        body, grid=grid,
        in_specs=([pl.BlockSpec((bn, x.shape[1]), lambda i: (i, 0))
                   for x in xs]
                  + [pl.BlockSpec(w.shape, lambda i: (0, 0))
                     for _, w in dots]
                  + [pl.BlockSpec((1, H), lambda i: (0, 0))]),
        out_specs=pl.BlockSpec((bn, H), lambda i: (i, 0)),
        out_shape=jax.ShapeDtypeStruct((np_rows, H), jnp.float32),
    )(*xs, *[w for _, w in dots], bias)


def _pool_combine(xs, dots, bias, ids3, wt, b, bn=2048):
    np_rows = xs[0].shape[0]
    ngrid = np_rows // bn
    k = len(xs)

    def body(*refs):
        x_refs = refs[:k]
        w_refs = refs[k:k + len(dots)]
        b_ref = refs[k + len(dots)]
        ids_ref = refs[k + len(dots) + 1]
        wo_ref = refs[k + len(dots) + 2]
        bo_ref = refs[k + len(dots) + 3]
        o_ref = refs[k + len(dots) + 4]
        acc = refs[k + len(dots) + 5]
        cnt = refs[k + len(dots) + 6]
        i = pl.program_id(0)

        @pl.when(i == 0)
        def _():
            acc[...] = jnp.zeros_like(acc)
            cnt[...] = jnp.zeros_like(cnt)

        xb = [_bf(x_ref[...]) for x_ref in x_refs]
        y = jnp.broadcast_to(b_ref[...], (bn, H))
        for (xi, _), w_ref in zip(dots, w_refs):
            y = y + _bdot(xb[xi], w_ref[...])
        y = jnp.maximum(y, 0.0)
        ids = ids_ref[0]
        gids = lax.broadcasted_iota(jnp.int32, (NUM_GRAPHS, bn), 0)
        oh = (gids == ids).astype(jnp.float32)
        acc[...] += jnp.dot(oh, y, precision=lax.Precision.HIGHEST)
        cnt[...] += jnp.sum(oh, axis=1, keepdims=True)

        @pl.when(i == ngrid - 1)
        def _():
            pooled = acc[...] / jnp.maximum(cnt[...], 1.0)
            o_ref[...] = _bdot(_bf(pooled), _bf(wo_ref[...])) + bo_ref[...]

    return pl.pallas_call(
        body, grid=(ngrid,),
        in_specs=([pl.BlockSpec((bn, x.shape[1]), lambda i: (i, 0))
                   for x in xs]
                  + [pl.BlockSpec(w.shape, lambda i: (0, 0))
                     for _, w in dots]
                  + [pl.BlockSpec((1, H), lambda i: (0, 0)),
                     pl.BlockSpec((1, 1, bn), lambda i: (i, 0, 0)),
                     pl.BlockSpec(wt.shape, lambda i: (0, 0)),
                     pl.BlockSpec((1, 1), lambda i: (0, 0))]),
        out_specs=pl.BlockSpec((NUM_GRAPHS, 1), lambda i: (0, 0)),
        out_shape=jax.ShapeDtypeStruct((NUM_GRAPHS, 1), jnp.float32),
        scratch_shapes=[
            pltpu.VMEM((NUM_GRAPHS, H), jnp.float32),
            pltpu.VMEM((NUM_GRAPHS, 1), jnp.float32),
        ],
    )(*xs, *[w for _, w in dots], bias, ids3, wt, b)


def _sc_layer(dst_specs, rel_erows, zero, zs, srcs, dsts):
    ntypes = len(dst_specs)
    nrels = len(zs)
    mesh = plsc.VectorSubcoreMesh(core_axis_name="c", subcore_axis_name="s",
                                  num_cores=NCORE, num_subcores=NSUB)

    @functools.partial(
        pl.kernel,
        out_type=[jax.ShapeDtypeStruct((sp[0], H), jnp.float32)
                  for sp in dst_specs],
        mesh=mesh,
        scratch_types=[
            pltpu.VMEM_SHARED((_ACC_ROWS, L), jnp.float32),
            pltpu.VMEM((_EROWS_MAX, 128), jnp.int32),
            pltpu.VMEM((_EROWS_MAX, 128), jnp.int32),
            pltpu.VMEM((8 * 128, L), jnp.float32),
            pltpu.SemaphoreType.DMA,
            pltpu.SemaphoreType.DMA,
        ],
        compiler_params=pltpu.CompilerParams(use_tc_tiling_on_sc=False),
    )
    def kfn(*refs):
        zero_ref = refs[0]
        z_refs = refs[1:1 + nrels]
        s_refs = refs[1 + nrels:1 + 2 * nrels]
        d_refs = refs[1 + 2 * nrels:1 + 3 * nrels]
        out_refs = refs[1 + 3 * nrels:1 + 3 * nrels + ntypes]
        acc, sidx, didx, rows, gsem, ssem = refs[1 + 3 * nrels + ntypes:]

        cid = lax.axis_index("c")
        sid = lax.axis_index("s")

        for ti, (nt_pad, rel_ids) in enumerate(dst_specs):
            rpw = nt_pad // NSUB
            for cc in range(CHUNKS_PER_CORE):
                ch = cid * CHUNKS_PER_CORE + cc
                pltpu.sync_copy(zero_ref.at[pl.ds(0, rpw)],
                                acc.at[pl.ds(sid * rpw, rpw)])
                plsc.subcore_barrier()
                for r in rel_ids:
                    erows = rel_erows[r]
                    stride = s_refs[r].shape[1] // NSUB
                    ki = _inner_k(erows)
                    nouter = erows // ki
                    pltpu.sync_copy(
                        s_refs[r].at[ch].at[pl.ds(sid * stride, stride)],
                        sidx.at[pl.ds(0, stride)])
                    pltpu.sync_copy(d_refs[r].at[pl.ds(sid * stride, stride)],
                                    didx.at[pl.ds(0, stride)])

                    def outer(o, _, r=r, ki=ki):
                        gds, sds = [], []
                        for j in range(ki):
                            gds.append(pltpu.async_copy(
                                z_refs[r].at[sidx.at[o * ki + j]],
                                rows.at[pl.ds(j * 128, 128)], gsem))
                        for dsc in gds:
                            dsc.wait()
                        for j in range(ki):
                            sds.append(pltpu.async_copy(
                                rows.at[pl.ds(j * 128, 128)],
                                acc.at[didx.at[o * ki + j]], ssem, add=True))
                        for dsc in sds:
                            dsc.wait()
                        return 0

                    lax.fori_loop(0, nouter, outer, 0)
                plsc.subcore_barrier()
                pltpu.sync_copy(
                    acc.at[pl.ds(sid * rpw, rpw)],
                    out_refs[ti].at[pl.ds(sid * rpw, rpw), pl.ds(ch * L, L)])
                plsc.subcore_barrier()

    return kfn(zero, *zs, *srcs, *dsts)


def kernel(x_operator, x_table, x_column, x_predicate, params,
           edge_index_scannedby, edge_index_filters, edge_index_outputby,
           edge_index_connects, edge_index_calledby,
           edge_index_selfloop_table, edge_index_selfloop_column,
           batch_operator):
    xs = {'operator': x_operator, 'table': x_table,
          'column': x_column, 'predicate': x_predicate}
    edges = {'scannedby': edge_index_scannedby, 'filters': edge_index_filters,
             'outputby': edge_index_outputby, 'connects': edge_index_connects,
             'calledby': edge_index_calledby,
             'selfloop_table': edge_index_selfloop_table,
             'selfloop_column': edge_index_selfloop_column}
    p = params

    xp = {t: jnp.pad(xs[t], ((0, _NPAD[t] - _N[t]), (0, 0))) for t in _NTYPES}
    srcp, dstp, erows_d = {}, {}, {}
    for (st, dt, name) in _ETYPES:
        e = _ECNT[name]
        ep = _EPAD[name]
        erows = ep // (NSUB * 128)
        stride = _ceil_to(erows, 8)
        erows_d[name] = erows

        def _lay(v, fill):
            v = jnp.pad(v, (0, ep - e), constant_values=fill)
            v = v.reshape(NSUB, erows, 128)
            v = jnp.pad(v, ((0, 0), (0, stride - erows), (0, 0)),
                        constant_values=fill)
            return v.reshape(NSUB * stride, 128)

        s0 = _lay(edges[name][0], 0)
        srcp[name] = (s0[None] * NCH
                      + jnp.arange(NCH, dtype=jnp.int32)[:, None, None])
        dstp[name] = _lay(edges[name][1], _N[dt])

    max_rpw = max(_NPAD[t] for t in _NTYPES) // NSUB
    zero = jnp.zeros((max_rpw, L), jnp.float32)
    rel_order = [name for (_, _, name) in _ETYPES]
    erows_l = [erows_d[name] for name in rel_order]

    h0 = {}
    for t in _NTYPES:
        h0[t] = _combine([xp[t]], [(0, _bf(p['lin_%s_W' % t].T))],
                         p['lin_%s_b' % t].reshape(1, H), relu_out=False)

    def run_sc(rels):
        specs = [(_NPAD[dt], [ri]) for ri, (st, dt, name) in enumerate(rels)]
        return _sc_layer(specs, [erows_d[name] for (_, _, name) in rels],
                         zero,
                         [h_src[st] for (st, dt, name) in rels],
                         [srcp[name] for (_, _, name) in rels],
                         [dstp[name] for (_, _, name) in rels])

    h_src = {t: h0[t].reshape(_NPAD[t] * NCH, L) for t in _NTYPES}
    l1a = [(st, dt, name) for (st, dt, name) in _ETYPES if dt != 'operator']
    l1b = [(st, dt, name) for (st, dt, name) in _ETYPES if dt == 'operator']
    agg1 = dict(zip([name for (_, _, name) in l1a], run_sc(l1a)))
    agg1.update(zip([name for (_, _, name) in l1b], run_sc(l1b)))

    out1 = {}
    for t in ['table', 'column', 'predicate', 'operator']:
        rels_t = [name for (st, dt, name) in _ETYPES if dt == t]
        ins = [agg1[name] for name in rels_t] + [h0[t]]
        dots = [(i, _bf(p['c1_%s_Wrel' % name].T))
                for i, name in enumerate(rels_t)]
        dots += [(len(rels_t), _bf(p['c1_%s_Wroot' % name].T))
                 for name in rels_t]
        bias = sum(p['c1_%s_brel' % name] for name in rels_t).reshape(1, H)
        out1[t] = _combine(ins, dots, bias, relu_out=True)

    l2_rels = ([(st, dt, name) for (st, dt, name) in _ETYPES
                if dt == 'operator' and st != 'operator']
               + [(st, dt, name) for (st, dt, name) in _ETYPES
                  if dt == 'operator' and st == 'operator'])
    h_src = {t: out1[t].reshape(_NPAD[t] * NCH, L) for t in _NTYPES}
    l2a = l2_rels[:-1]
    l2b = l2_rels[-1:]
    aggs2 = list(run_sc(l2a)) + list(run_sc(l2b))

    ins2 = list(aggs2) + [out1['operator']]
    dots2 = [(i, _bf(p['c2_%s_Wrel' % name].T))
             for i, (_, _, name) in enumerate(l2_rels)]
    dots2 += [(len(l2_rels), _bf(p['c2_%s_Wroot' % name].T))
              for (_, _, name) in l2_rels]
    bias2 = sum(p['c2_%s_brel' % name] for (_, _, name) in l2_rels
                ).reshape(1, H)
    ids = jnp.pad(batch_operator, (0, _NPAD['operator'] - _N['operator']),
                  constant_values=NUM_GRAPHS + 1)
    ids3 = ids.reshape(_NPAD['operator'] // 2048, 1, 2048)
    res = _pool_combine(ins2, dots2, bias2, ids3, p['lin_out_W'].T,
                        p['lin_out_b'].reshape(1, 1))
    return res.reshape(NUM_GRAPHS)


# speedup vs baseline: 1.0622x; 1.0332x over previous
import functools

import jax
import jax.numpy as jnp
from jax import lax
from jax.experimental import pallas as pl
from jax.experimental.pallas import tpu as pltpu
from jax.experimental.pallas import tpu_sc as plsc

H = 128
L = 16
NCH = H // L
NSUB = 16
NCORE = 2
CHUNKS_PER_CORE = NCH // NCORE

_N = {'operator': 100000, 'table': 50000, 'column': 100000, 'predicate': 80000}
_ETYPES = [('table', 'operator', 'scannedby'),
           ('predicate', 'operator', 'filters'),
           ('column', 'operator', 'outputby'),
           ('column', 'predicate', 'connects'),
           ('operator', 'operator', 'calledby'),
           ('table', 'table', 'selfloop_table'),
           ('column', 'column', 'selfloop_column')]
_ECNT = {'scannedby': 100000, 'filters': 100000, 'outputby': 100000,
         'connects': 100000, 'calledby': 100000, 'selfloop_table': 50000,
         'selfloop_column': 100000}
NUM_GRAPHS = 64
_NTYPES = ['operator', 'table', 'column', 'predicate']


def _ceil_to(x, m):
    return (x + m - 1) // m * m


def _bf(a):
    return a.astype(jnp.bfloat16)


def _bdot(a_bf, b_bf):
    return jnp.dot(a_bf, b_bf, preferred_element_type=jnp.float32)


_NPAD = {t: _ceil_to(_N[t] + 1, 2048) for t in _NTYPES}
_EPAD = {n: _ceil_to(_ECNT[n], 2048) for n in _ECNT}
_ACC_ROWS = max(_NPAD.values())
_EROWS_MAX = _ceil_to(max(_EPAD.values()) // (NSUB * 128), 8)


def _inner_k(nrows):
    for k in (8, 7, 5, 6, 4, 3, 2):
        if nrows % k == 0:
            return k
    return 1


def _combine(xs, dots, bias, relu_out=True, bn=2048):
    np_rows = xs[0].shape[0]
    grid = (np_rows // bn,)
    k = len(xs)

    def body(*refs):
        x_refs = refs[:k]
        w_refs = refs[k:k + len(dots)]
        b_ref = refs[k + len(dots)]
        o_ref = refs[k + len(dots) + 1]
        xb = [_bf(x_ref[...]) for x_ref in x_refs]
        y = jnp.broadcast_to(b_ref[...], (bn, H))
        for (xi, _), w_ref in zip(dots, w_refs):
            y = y + _bdot(xb[xi], w_ref[...])
        if relu_out:
            y = jnp.maximum(y, 0.0)
        o_ref[...] = y

    return pl.pallas_call(
        body, grid=grid,
        in_specs=([pl.BlockSpec((bn, x.shape[1]), lambda i: (i, 0))
                   for x in xs]
                  + [pl.BlockSpec(w.shape, lambda i: (0, 0))
                     for _, w in dots]
                  + [pl.BlockSpec((1, H), lambda i: (0, 0))]),
        out_specs=pl.BlockSpec((bn, H), lambda i: (i, 0)),
        out_shape=jax.ShapeDtypeStruct((np_rows, H), jnp.float32),
    )(*xs, *[w for _, w in dots], bias)


def _pool_combine(xs, dots, bias, ids3, wt, b, bn=2048):
    np_rows = xs[0].shape[0]
    ngrid = np_rows // bn
    k = len(xs)

    def body(*refs):
        x_refs = refs[:k]
        w_refs = refs[k:k + len(dots)]
        b_ref = refs[k + len(dots)]
        ids_ref = refs[k + len(dots) + 1]
        wo_ref = refs[k + len(dots) + 2]
        bo_ref = refs[k + len(dots) + 3]
        o_ref = refs[k + len(dots) + 4]
        acc = refs[k + len(dots) + 5]
        cnt = refs[k + len(dots) + 6]
        i = pl.program_id(0)

        @pl.when(i == 0)
        def _():
            acc[...] = jnp.zeros_like(acc)
            cnt[...] = jnp.zeros_like(cnt)

        xb = [_bf(x_ref[...]) for x_ref in x_refs]
        y = jnp.broadcast_to(b_ref[...], (bn, H))
        for (xi, _), w_ref in zip(dots, w_refs):
            y = y + _bdot(xb[xi], w_ref[...])
        y = jnp.maximum(y, 0.0)
        ids = ids_ref[0]
        gids = lax.broadcasted_iota(jnp.int32, (NUM_GRAPHS, bn), 0)
        oh = (gids == ids).astype(jnp.float32)
        acc[...] += jnp.dot(oh, y, precision=lax.Precision.HIGHEST)
        cnt[...] += jnp.sum(oh, axis=1, keepdims=True)

        @pl.when(i == ngrid - 1)
        def _():
            pooled = acc[...] / jnp.maximum(cnt[...], 1.0)
            o_ref[...] = _bdot(_bf(pooled), _bf(wo_ref[...])) + bo_ref[...]

    return pl.pallas_call(
        body, grid=(ngrid,),
        in_specs=([pl.BlockSpec((bn, x.shape[1]), lambda i: (i, 0))
                   for x in xs]
                  + [pl.BlockSpec(w.shape, lambda i: (0, 0))
                     for _, w in dots]
                  + [pl.BlockSpec((1, H), lambda i: (0, 0)),
                     pl.BlockSpec((1, 1, bn), lambda i: (i, 0, 0)),
                     pl.BlockSpec(wt.shape, lambda i: (0, 0)),
                     pl.BlockSpec((1, 1), lambda i: (0, 0))]),
        out_specs=pl.BlockSpec((NUM_GRAPHS, 1), lambda i: (0, 0)),
        out_shape=jax.ShapeDtypeStruct((NUM_GRAPHS, 1), jnp.float32),
        scratch_shapes=[
            pltpu.VMEM((NUM_GRAPHS, H), jnp.float32),
            pltpu.VMEM((NUM_GRAPHS, 1), jnp.float32),
        ],
    )(*xs, *[w for _, w in dots], bias, ids3, wt, b)


def _sc_layer(dst_specs, rel_erows, zero, zs, srcs, dsts):
    ntypes = len(dst_specs)
    nrels = len(zs)
    mesh = plsc.VectorSubcoreMesh(core_axis_name="c", subcore_axis_name="s",
                                  num_cores=NCORE, num_subcores=NSUB)

    @functools.partial(
        pl.kernel,
        out_type=[jax.ShapeDtypeStruct((sp[0], H), jnp.float32)
                  for sp in dst_specs],
        mesh=mesh,
        scratch_types=[
            pltpu.VMEM_SHARED((_ACC_ROWS, L), jnp.float32),
            pltpu.VMEM((_EROWS_MAX, 128), jnp.int32),
            pltpu.VMEM((_EROWS_MAX, 128), jnp.int32),
            pltpu.VMEM((8 * 128, L), jnp.float32),
            pltpu.SemaphoreType.DMA,
            pltpu.SemaphoreType.DMA,
        ],
        compiler_params=pltpu.CompilerParams(use_tc_tiling_on_sc=False),
    )
    def kfn(*refs):
        zero_ref = refs[0]
        z_refs = refs[1:1 + nrels]
        s_refs = refs[1 + nrels:1 + 2 * nrels]
        d_refs = refs[1 + 2 * nrels:1 + 3 * nrels]
        out_refs = refs[1 + 3 * nrels:1 + 3 * nrels + ntypes]
        acc, sidx, didx, rows, gsem, ssem = refs[1 + 3 * nrels + ntypes:]

        cid = lax.axis_index("c")
        sid = lax.axis_index("s")

        for ti, (nt_pad, rel_ids) in enumerate(dst_specs):
            rpw = nt_pad // NSUB
            for cc in range(CHUNKS_PER_CORE):
                ch = cid * CHUNKS_PER_CORE + cc
                pltpu.sync_copy(zero_ref.at[pl.ds(0, rpw)],
                                acc.at[pl.ds(sid * rpw, rpw)])
                plsc.subcore_barrier()
                for r in rel_ids:
                    erows = rel_erows[r]
                    stride = s_refs[r].shape[1] // NSUB
                    ki = _inner_k(erows)
                    nouter = erows // ki
                    pltpu.sync_copy(
                        s_refs[r].at[ch].at[pl.ds(sid * stride, stride)],
                        sidx.at[pl.ds(0, stride)])
                    pltpu.sync_copy(d_refs[r].at[pl.ds(sid * stride, stride)],
                                    didx.at[pl.ds(0, stride)])

                    def outer(o, _, r=r, ki=ki):
                        gds, sds = [], []
                        for j in range(ki):
                            gds.append(pltpu.async_copy(
                                z_refs[r].at[sidx.at[o * ki + j]],
                                rows.at[pl.ds(j * 128, 128)], gsem))
                        for dsc in gds:
                            dsc.wait()
                        for j in range(ki):
                            sds.append(pltpu.async_copy(
                                rows.at[pl.ds(j * 128, 128)],
                                acc.at[didx.at[o * ki + j]], ssem, add=True))
                        for dsc in sds:
                            dsc.wait()
                        return 0

                    lax.fori_loop(0, nouter, outer, 0)
                plsc.subcore_barrier()
                pltpu.sync_copy(
                    acc.at[pl.ds(sid * rpw, rpw)],
                    out_refs[ti].at[pl.ds(sid * rpw, rpw), pl.ds(ch * L, L)])

    return kfn(zero, *zs, *srcs, *dsts)


def kernel(x_operator, x_table, x_column, x_predicate, params,
           edge_index_scannedby, edge_index_filters, edge_index_outputby,
           edge_index_connects, edge_index_calledby,
           edge_index_selfloop_table, edge_index_selfloop_column,
           batch_operator):
    xs = {'operator': x_operator, 'table': x_table,
          'column': x_column, 'predicate': x_predicate}
    edges = {'scannedby': edge_index_scannedby, 'filters': edge_index_filters,
             'outputby': edge_index_outputby, 'connects': edge_index_connects,
             'calledby': edge_index_calledby,
             'selfloop_table': edge_index_selfloop_table,
             'selfloop_column': edge_index_selfloop_column}
    p = params

    xp = {t: jnp.pad(xs[t], ((0, _NPAD[t] - _N[t]), (0, 0))) for t in _NTYPES}
    srcp, dstp, erows_d = {}, {}, {}
    for (st, dt, name) in _ETYPES:
        e = _ECNT[name]
        ep = _EPAD[name]
        erows = ep // (NSUB * 128)
        stride = _ceil_to(erows, 8)
        erows_d[name] = erows

        def _lay(v, fill):
            v = jnp.pad(v, (0, ep - e), constant_values=fill)
            v = v.reshape(NSUB, erows, 128)
            v = jnp.pad(v, ((0, 0), (0, stride - erows), (0, 0)),
                        constant_values=fill)
            return v.reshape(NSUB * stride, 128)

        s0 = _lay(edges[name][0], 0)
        srcp[name] = (s0[None] * NCH
                      + jnp.arange(NCH, dtype=jnp.int32)[:, None, None])
        dstp[name] = _lay(edges[name][1], _N[dt])

    max_rpw = max(_NPAD[t] for t in _NTYPES) // NSUB
    zero = jnp.zeros((max_rpw, L), jnp.float32)
    rel_order = [name for (_, _, name) in _ETYPES]
    erows_l = [erows_d[name] for name in rel_order]

    h0 = {}
    for t in _NTYPES:
        h0[t] = _combine([xp[t]], [(0, _bf(p['lin_%s_W' % t].T))],
                         p['lin_%s_b' % t].reshape(1, H), relu_out=False)

    def run_sc(rels):
        specs = [(_NPAD[dt], [ri]) for ri, (st, dt, name) in enumerate(rels)]
        return _sc_layer(specs, [erows_d[name] for (_, _, name) in rels],
                         zero,
                         [h_src[st] for (st, dt, name) in rels],
                         [srcp[name] for (_, _, name) in rels],
                         [dstp[name] for (_, _, name) in rels])

    h_src = {t: h0[t].reshape(_NPAD[t] * NCH, L) for t in _NTYPES}
    l1a = [(st, dt, name) for (st, dt, name) in _ETYPES if dt != 'operator']
    l1b = [(st, dt, name) for (st, dt, name) in _ETYPES if dt == 'operator']
    agg1 = dict(zip([name for (_, _, name) in l1a], run_sc(l1a)))
    agg1.update(zip([name for (_, _, name) in l1b], run_sc(l1b)))

    out1 = {}
    for t in ['table', 'column', 'predicate', 'operator']:
        rels_t = [name for (st, dt, name) in _ETYPES if dt == t]
        ins = [agg1[name] for name in rels_t] + [h0[t]]
        dots = [(i, _bf(p['c1_%s_Wrel' % name].T))
                for i, name in enumerate(rels_t)]
        dots += [(len(rels_t), _bf(p['c1_%s_Wroot' % name].T))
                 for name in rels_t]
        bias = sum(p['c1_%s_brel' % name] for name in rels_t).reshape(1, H)
        out1[t] = _combine(ins, dots, bias, relu_out=True)

    l2_rels = ([(st, dt, name) for (st, dt, name) in _ETYPES
                if dt == 'operator' and st != 'operator']
               + [(st, dt, name) for (st, dt, name) in _ETYPES
                  if dt == 'operator' and st == 'operator'])
    h_src = {t: out1[t].reshape(_NPAD[t] * NCH, L) for t in _NTYPES}
    l2a = l2_rels[:-1]
    l2b = l2_rels[-1:]
    aggs2 = list(run_sc(l2a)) + list(run_sc(l2b))

    ins2 = list(aggs2) + [out1['operator']]
    dots2 = [(i, _bf(p['c2_%s_Wrel' % name].T))
             for i, (_, _, name) in enumerate(l2_rels)]
    dots2 += [(len(l2_rels), _bf(p['c2_%s_Wroot' % name].T))
              for (_, _, name) in l2_rels]
    bias2 = sum(p['c2_%s_brel' % name] for (_, _, name) in l2_rels
                ).reshape(1, H)
    ids = jnp.pad(batch_operator, (0, _NPAD['operator'] - _N['operator']),
                  constant_values=NUM_GRAPHS + 1)
    ids3 = ids.reshape(_NPAD['operator'] // 2048, 1, 2048)
    res = _pool_combine(ins2, dots2, bias2, ids3, p['lin_out_W'].T,
                        p['lin_out_b'].reshape(1, 1))
    return res.reshape(NUM_GRAPHS)
